# baseline (speedup 1.0000x reference)
import functools

import jax
import jax.numpy as jnp
from jax import lax
from jax.experimental import pallas as pl
from jax.experimental.pallas import tpu as pltpu
from jax.experimental.pallas import tpu_sc as plsc

_NBUF = 8
_BLK = 16
_IDX = 128


@functools.lru_cache(maxsize=None)
def _make_gather(B, V, D):
    info = plsc.get_sparse_core_info()
    nw = info.num_cores * info.num_subcores
    b_per_w = B // nw
    n_tiles = V // 128
    v_last = n_tiles * 128
    last_w = V - v_last
    max_tile = n_tiles - 1
    mesh = plsc.VectorSubcoreMesh(core_axis_name="c", subcore_axis_name="s")

    @functools.partial(
        pl.kernel,
        mesh=mesh,
        out_type=(
            jax.ShapeDtypeStruct((B, D), jnp.float32),
            jax.ShapeDtypeStruct((B, D), jnp.float32),
        ),
        scratch_types=[
            pltpu.VMEM((b_per_w,), jnp.int32),
            pltpu.VMEM((_NBUF, D, 128), jnp.float32),
            pltpu.VMEM((D, last_w or 1), jnp.float32),
            pltpu.VMEM((2, _BLK, D), jnp.float32),
        ] + [pltpu.SemaphoreType.DMA] * (_NBUF + 2),
        compiler_params=pltpu.CompilerParams(
            use_tc_tiling_on_sc=True, needs_layout_passes=False),
    )
    def gather(ids_hbm, wst_hbm, wat_hbm, out_s_hbm, out_a_hbm,
               idx_v, win_v, win_t, rows_v, *sems):
        semw = sems[_NBUF:]
        wid = lax.axis_index("s") * info.num_cores + lax.axis_index("c")
        base = wid * b_per_w
        pltpu.sync_copy(ids_hbm.at[pl.ds(base, b_per_w)], idx_v)
        iota16 = lax.iota(jnp.int32, 16)
        def lanes(vec):
            tiles = jnp.minimum(vec // 128, max_tile)
            cols = tiles * 128
            rins = jnp.minimum(vec - cols, 127)
            return tiles, cols, rins

        def run_table(w_hbm, out_hbm, first):
            if last_w:
                pltpu.sync_copy(w_hbm.at[:, pl.ds(v_last, last_w)], win_t)

            def issue(col, b):
                col = pl.multiple_of(col, 128)
                pltpu.async_copy(
                    w_hbm.at[:, pl.ds(col, 128)], win_v.at[b], sems[b])

            def maybe_issue(m, tiles, cols):
                b = m % _NBUF
                if m % _NBUF == 0:
                    issue(cols[m], b)
                else:
                    @pl.when(tiles[m] != tiles[m - 1])
                    def _():
                        issue(cols[m], b)

            def extract(p, j, vec, tiles, rins, s_prev):
                b = j % _NBUF
                if j % _NBUF == 0:
                    pltpu.make_async_copy(
                        w_hbm.at[:, pl.ds(0, 128)], win_v.at[b],
                        sems[b]).wait()
                    s = jnp.int32(b)
                else:
                    cond = tiles[j] != tiles[j - 1]

                    @pl.when(cond)
                    def _():
                        pltpu.make_async_copy(
                            w_hbm.at[:, pl.ds(0, 128)], win_v.at[b],
                            sems[b]).wait()

                    s = jnp.where(cond, jnp.int32(b), s_prev)
                slot16 = jnp.full((16,), s, jnp.int32)
                cols16 = jnp.full((16,), rins[j], jnp.int32)
                for k in range(D // 16):
                    vals = plsc.load_gather(
                        win_v, [slot16, iota16 + k * 16, cols16])
                    rows_v[p, j, pl.ds(k * 16, 16)] = vals
                if last_w:
                    @pl.when(vec[j] >= v_last)
                    def _():
                        c2 = jnp.full((16,), vec[j] - v_last, jnp.int32)
                        for k in range(D // 16):
                            vals = plsc.load_gather(
                                win_t, [iota16 + k * 16, c2])
                            rows_v[p, j, pl.ds(k * 16, 16)] = vals
                return s

            def block(io, p, nxt_io, skip_wait=False):
                if not skip_wait:
                    pltpu.make_async_copy(
                        rows_v.at[p], out_hbm.at[pl.ds(base, _BLK)],
                        semw[p]).wait()
                vec = idx_v[pl.ds(io, _BLK)]
                tiles, cols, rins = lanes(vec)
                if nxt_io is not None:
                    vecn = idx_v[pl.ds(nxt_io, _BLK)]
                    tilesn, colsn, _ = lanes(vecn)
                s = jnp.int32(0)
                for j in range(_BLK):
                    s = extract(p, j, vec, tiles, rins, s)
                    m = j + _NBUF
                    if m < _BLK:
                        maybe_issue(m, tiles, cols)
                    elif nxt_io is not None:
                        maybe_issue(m - _BLK, tilesn, colsn)
                pltpu.async_copy(
                    rows_v.at[p], out_hbm.at[pl.ds(base + io, _BLK)],
                    semw[p])

            vec0 = idx_v[pl.ds(0, _BLK)]
            t0, c0, _ = lanes(vec0)
            for j in range(_NBUF):
                maybe_issue(j, t0, c0)

            block(0, 0, _BLK, skip_wait=first)
            block(_BLK, 1, 2 * _BLK, skip_wait=first)

            def steady(i0):
                block(i0, 0, i0 + _BLK)
                block(i0 + _BLK, 1, i0 + 2 * _BLK)

            pl.loop(2 * _BLK, b_per_w - 2 * _BLK, step=2 * _BLK)(steady)

            block(b_per_w - 2 * _BLK, 0, b_per_w - _BLK)
            block(b_per_w - _BLK, 1, None)

        run_table(wst_hbm, out_s_hbm, True)
        run_table(wat_hbm, out_a_hbm, False)

        for p in range(2):
            pltpu.make_async_copy(
                rows_v.at[p], out_a_hbm.at[pl.ds(base, _BLK)],
                semw[p]).wait()

    return gather


@functools.lru_cache(maxsize=None)
def _make_unpermute(B, D):
    info = plsc.get_sparse_core_info()
    nw = info.num_cores * info.num_subcores
    b_per_w = B // nw
    n_desc = b_per_w // _IDX
    mesh = plsc.VectorSubcoreMesh(core_axis_name="c", subcore_axis_name="s")

    @functools.partial(
        pl.kernel,
        mesh=mesh,
        out_type=(
            jax.ShapeDtypeStruct((B, D), jnp.float32),
            jax.ShapeDtypeStruct((B, D), jnp.float32),
        ),
        scratch_types=[
            pltpu.VMEM((n_desc, _IDX), jnp.int32),
            pltpu.VMEM((b_per_w, D), jnp.float32),
            pltpu.SemaphoreType.DMA,
        ],
        compiler_params=pltpu.CompilerParams(use_tc_tiling_on_sc=False),
    )
    def unpermute(rows_s_hbm, rows_a_hbm, perm_hbm, out_s_hbm, out_a_hbm,
                  idx_v, rows_v, sem):
        wid = lax.axis_index("s") * info.num_cores + lax.axis_index("c")
        base = wid * b_per_w
        pltpu.sync_copy(perm_hbm.at[wid], idx_v)

        def scatter(rows_hbm, out_hbm):
            pltpu.sync_copy(rows_hbm.at[pl.ds(base, b_per_w)], rows_v)
            copies = []
            for j in range(n_desc):
                src = rows_v.at[pl.ds(j * _IDX, _IDX)]
                copies.append(
                    pltpu.async_copy(src, out_hbm.at[idx_v.at[j]], sem))
            for c in copies:
                c.wait()

        scatter(rows_s_hbm, out_s_hbm)
        scatter(rows_a_hbm, out_a_hbm)

    return unpermute


def kernel(instance_ids, W_shape, W_appearance):
    (B,) = instance_ids.shape
    V, D = W_shape.shape
    info = plsc.get_sparse_core_info()
    nw = info.num_cores * info.num_subcores
    ids = instance_ids.astype(jnp.int32)
    sid, perm = ids, jnp.arange(B, dtype=jnp.int32)
    rows_s, rows_a = _make_gather(B, V, D)(sid, W_shape.T, W_appearance.T)
    perm3 = perm.reshape(nw, (B // nw) // _IDX, _IDX)
    out_s, out_a = _make_unpermute(B, D)(rows_s, rows_a, perm3)
    return (out_s, out_a)


# speedup vs baseline: 1.4020x; 1.4020x over previous
import functools

import jax
import jax.numpy as jnp
from jax import lax
from jax.experimental import pallas as pl
from jax.experimental.pallas import tpu as pltpu
from jax.experimental.pallas import tpu_sc as plsc

_NBUF = 8
_BLK = 16
_IDX = 128


@functools.lru_cache(maxsize=None)
def _make_gather(B, V, D):
    info = plsc.get_sparse_core_info()
    nw = info.num_cores * info.num_subcores
    b_per_w = B // nw
    n_tiles = V // 128
    v_last = n_tiles * 128
    last_w = V - v_last
    max_tile = n_tiles - 1
    mesh = plsc.VectorSubcoreMesh(core_axis_name="c", subcore_axis_name="s")

    @functools.partial(
        pl.kernel,
        mesh=mesh,
        out_type=(
            jax.ShapeDtypeStruct((B, D), jnp.float32),
            jax.ShapeDtypeStruct((B, D), jnp.float32),
        ),
        scratch_types=[
            pltpu.VMEM((b_per_w,), jnp.int32),
            pltpu.VMEM((_NBUF, D, 128), jnp.float32),
            pltpu.VMEM((D, last_w or 1), jnp.float32),
            pltpu.VMEM((2, _BLK, D), jnp.float32),
        ] + [pltpu.SemaphoreType.DMA] * (_NBUF + 2),
        compiler_params=pltpu.CompilerParams(
            use_tc_tiling_on_sc=True, needs_layout_passes=False),
    )
    def gather(ids_hbm, wst_hbm, wat_hbm, out_s_hbm, out_a_hbm,
               idx_v, win_v, win_t, rows_v, *sems):
        semw = sems[_NBUF:]
        wid = lax.axis_index("s") * info.num_cores + lax.axis_index("c")
        base = wid * b_per_w
        pltpu.sync_copy(ids_hbm.at[pl.ds(base, b_per_w)], idx_v)
        iota16 = lax.iota(jnp.int32, 16)
        def lanes(vec):
            tiles = jnp.minimum(vec // 128, max_tile)
            cols = tiles * 128
            rins = jnp.minimum(vec - cols, 127)
            return tiles, cols, rins

        def run_table(w_hbm, out_hbm, first):
            if last_w:
                pltpu.sync_copy(w_hbm.at[:, pl.ds(v_last, last_w)], win_t)

            def issue(col, b):
                col = pl.multiple_of(col, 128)
                pltpu.async_copy(
                    w_hbm.at[:, pl.ds(col, 128)], win_v.at[b], sems[b])

            def maybe_issue(m, tiles, cols):
                b = m % _NBUF
                if m % _NBUF == 0:
                    issue(cols[m], b)
                else:
                    @pl.when(tiles[m] != tiles[m - 1])
                    def _():
                        issue(cols[m], b)

            def extract(p, j, vec, tiles, rins, s_prev):
                b = j % _NBUF
                if j % _NBUF == 0:
                    pltpu.make_async_copy(
                        w_hbm.at[:, pl.ds(0, 128)], win_v.at[b],
                        sems[b]).wait()
                    s = jnp.int32(b)
                else:
                    cond = tiles[j] != tiles[j - 1]

                    @pl.when(cond)
                    def _():
                        pltpu.make_async_copy(
                            w_hbm.at[:, pl.ds(0, 128)], win_v.at[b],
                            sems[b]).wait()

                    s = jnp.where(cond, jnp.int32(b), s_prev)
                slot16 = jnp.full((16,), s, jnp.int32)
                cols16 = jnp.full((16,), rins[j], jnp.int32)
                for k in range(D // 16):
                    vals = plsc.load_gather(
                        win_v, [slot16, iota16 + k * 16, cols16])
                    rows_v[p, j, pl.ds(k * 16, 16)] = vals
                if last_w:
                    @pl.when(vec[j] >= v_last)
                    def _():
                        c2 = jnp.full((16,), vec[j] - v_last, jnp.int32)
                        for k in range(D // 16):
                            vals = plsc.load_gather(
                                win_t, [iota16 + k * 16, c2])
                            rows_v[p, j, pl.ds(k * 16, 16)] = vals
                return s

            def block(io, p, nxt_io, skip_wait=False):
                if not skip_wait:
                    pltpu.make_async_copy(
                        rows_v.at[p], out_hbm.at[pl.ds(base, _BLK)],
                        semw[p]).wait()
                vec = idx_v[pl.ds(io, _BLK)]
                tiles, cols, rins = lanes(vec)
                if nxt_io is not None:
                    vecn = idx_v[pl.ds(nxt_io, _BLK)]
                    tilesn, colsn, _ = lanes(vecn)
                s = jnp.int32(0)
                for j in range(_BLK):
                    s = extract(p, j, vec, tiles, rins, s)
                    m = j + _NBUF
                    if m < _BLK:
                        maybe_issue(m, tiles, cols)
                    elif nxt_io is not None:
                        maybe_issue(m - _BLK, tilesn, colsn)
                pltpu.async_copy(
                    rows_v.at[p], out_hbm.at[pl.ds(base + io, _BLK)],
                    semw[p])

            vec0 = idx_v[pl.ds(0, _BLK)]
            t0, c0, _ = lanes(vec0)
            for j in range(_NBUF):
                maybe_issue(j, t0, c0)

            block(0, 0, _BLK, skip_wait=first)
            block(_BLK, 1, 2 * _BLK, skip_wait=first)

            def steady(i0):
                block(i0, 0, i0 + _BLK)
                block(i0 + _BLK, 1, i0 + 2 * _BLK)

            pl.loop(2 * _BLK, b_per_w - 2 * _BLK, step=2 * _BLK)(steady)

            block(b_per_w - 2 * _BLK, 0, b_per_w - _BLK)
            block(b_per_w - _BLK, 1, None)

        run_table(wst_hbm, out_s_hbm, True)
        run_table(wat_hbm, out_a_hbm, False)

        for p in range(2):
            pltpu.make_async_copy(
                rows_v.at[p], out_a_hbm.at[pl.ds(base, _BLK)],
                semw[p]).wait()

    return gather


@functools.lru_cache(maxsize=None)
def _make_unpermute(B, D):
    info = plsc.get_sparse_core_info()
    nw = info.num_cores * info.num_subcores
    b_per_w = B // nw
    n_desc = b_per_w // _IDX
    mesh = plsc.VectorSubcoreMesh(core_axis_name="c", subcore_axis_name="s")

    @functools.partial(
        pl.kernel,
        mesh=mesh,
        out_type=(
            jax.ShapeDtypeStruct((B, D), jnp.float32),
            jax.ShapeDtypeStruct((B, D), jnp.float32),
        ),
        scratch_types=[
            pltpu.VMEM((n_desc, _IDX), jnp.int32),
            pltpu.VMEM((b_per_w, D), jnp.float32),
            pltpu.SemaphoreType.DMA,
        ],
        compiler_params=pltpu.CompilerParams(use_tc_tiling_on_sc=False),
    )
    def unpermute(rows_s_hbm, rows_a_hbm, perm_hbm, out_s_hbm, out_a_hbm,
                  idx_v, rows_v, sem):
        wid = lax.axis_index("s") * info.num_cores + lax.axis_index("c")
        base = wid * b_per_w
        pltpu.sync_copy(perm_hbm.at[wid], idx_v)

        def scatter(rows_hbm, out_hbm):
            pltpu.sync_copy(rows_hbm.at[pl.ds(base, b_per_w)], rows_v)
            copies = []
            for j in range(n_desc):
                src = rows_v.at[pl.ds(j * _IDX, _IDX)]
                copies.append(
                    pltpu.async_copy(src, out_hbm.at[idx_v.at[j]], sem))
            for c in copies:
                c.wait()

        scatter(rows_s_hbm, out_s_hbm)
        scatter(rows_a_hbm, out_a_hbm)

    return unpermute


def kernel(instance_ids, W_shape, W_appearance):
    (B,) = instance_ids.shape
    V, D = W_shape.shape
    info = plsc.get_sparse_core_info()
    nw = info.num_cores * info.num_subcores
    ids = instance_ids.astype(jnp.int32)
    if (V // 128) < (1 << 17) and B & (B - 1) == 0 and B <= (1 << 14):
        pos_bits = B.bit_length() - 1
        keys = jnp.sort(((ids // 128) << pos_bits)
                        | jnp.arange(B, dtype=jnp.int32))
        perm = keys & (B - 1)
        sid = jnp.take(ids, perm)
    else:
        sid, perm = lax.sort_key_val(ids, jnp.arange(B, dtype=jnp.int32))
    rows_s, rows_a = _make_gather(B, V, D)(sid, W_shape.T, W_appearance.T)
    perm3 = perm.reshape(nw, (B // nw) // _IDX, _IDX)
    out_s, out_a = _make_unpermute(B, D)(rows_s, rows_a, perm3)
    return (out_s, out_a)


# speedup vs baseline: 1.4622x; 1.0430x over previous
import functools

import jax
import jax.numpy as jnp
from jax import lax
from jax.experimental import pallas as pl
from jax.experimental.pallas import tpu as pltpu
from jax.experimental.pallas import tpu_sc as plsc

_NBUF = 8
_BLK = 16
_IDX = 128


@functools.lru_cache(maxsize=None)
def _make_gather(B, V, D):
    info = plsc.get_sparse_core_info()
    nw = info.num_cores * info.num_subcores
    b_per_w = B // nw
    n_tiles = V // 128
    v_last = n_tiles * 128
    last_w = V - v_last
    max_tile = n_tiles - 1
    mesh = plsc.VectorSubcoreMesh(core_axis_name="c", subcore_axis_name="s")

    @functools.partial(
        pl.kernel,
        mesh=mesh,
        out_type=(
            jax.ShapeDtypeStruct((B, D), jnp.float32),
            jax.ShapeDtypeStruct((B, D), jnp.float32),
        ),
        scratch_types=[
            pltpu.VMEM((b_per_w,), jnp.int32),
            pltpu.VMEM((_NBUF, D, 128), jnp.float32),
            pltpu.VMEM((D, last_w or 1), jnp.float32),
            pltpu.VMEM((2, _BLK, D), jnp.float32),
        ] + [pltpu.SemaphoreType.DMA] * (_NBUF + 2),
        compiler_params=pltpu.CompilerParams(
            use_tc_tiling_on_sc=True, needs_layout_passes=False),
    )
    def gather(ids_hbm, wst_hbm, wat_hbm, out_s_hbm, out_a_hbm,
               idx_v, win_v, win_t, rows_v, *sems):
        semw = sems[_NBUF:]
        wid = lax.axis_index("s") * info.num_cores + lax.axis_index("c")
        base = wid * b_per_w
        pltpu.sync_copy(ids_hbm.at[pl.ds(base, b_per_w)], idx_v)
        iota16 = lax.iota(jnp.int32, 16)
        def lanes(vec):
            tiles = jnp.minimum(vec // 128, max_tile)
            cols = tiles * 128
            rins = jnp.minimum(vec - cols, 127)
            return tiles, cols, rins

        def run_table(w_hbm, out_hbm, first):
            if last_w:
                pltpu.sync_copy(w_hbm.at[:, pl.ds(v_last, last_w)], win_t)

            def issue(col, b):
                col = pl.multiple_of(col, 128)
                pltpu.async_copy(
                    w_hbm.at[:, pl.ds(col, 128)], win_v.at[b], sems[b])

            def maybe_issue(m, tiles, cols):
                b = m % _NBUF
                if m % _NBUF == 0:
                    issue(cols[m], b)
                else:
                    @pl.when(tiles[m] != tiles[m - 1])
                    def _():
                        issue(cols[m], b)

            def extract(p, j, vec, tiles, rins, s_prev):
                b = j % _NBUF
                if j % _NBUF == 0:
                    pltpu.make_async_copy(
                        w_hbm.at[:, pl.ds(0, 128)], win_v.at[b],
                        sems[b]).wait()
                    s = jnp.int32(b)
                else:
                    cond = tiles[j] != tiles[j - 1]

                    @pl.when(cond)
                    def _():
                        pltpu.make_async_copy(
                            w_hbm.at[:, pl.ds(0, 128)], win_v.at[b],
                            sems[b]).wait()

                    s = jnp.where(cond, jnp.int32(b), s_prev)
                slot16 = jnp.full((16,), s, jnp.int32)
                cols16 = jnp.full((16,), rins[j], jnp.int32)
                for k in range(D // 16):
                    vals = plsc.load_gather(
                        win_v, [slot16, iota16 + k * 16, cols16])
                    rows_v[p, j, pl.ds(k * 16, 16)] = vals
                if last_w:
                    @pl.when(vec[j] >= v_last)
                    def _():
                        c2 = jnp.full((16,), vec[j] - v_last, jnp.int32)
                        for k in range(D // 16):
                            vals = plsc.load_gather(
                                win_t, [iota16 + k * 16, c2])
                            rows_v[p, j, pl.ds(k * 16, 16)] = vals
                return s

            def block(io, p, nxt_io, skip_wait=False):
                if not skip_wait:
                    pltpu.make_async_copy(
                        rows_v.at[p], out_hbm.at[pl.ds(base, _BLK)],
                        semw[p]).wait()
                vec = idx_v[pl.ds(io, _BLK)]
                tiles, cols, rins = lanes(vec)
                if nxt_io is not None:
                    vecn = idx_v[pl.ds(nxt_io, _BLK)]
                    tilesn, colsn, _ = lanes(vecn)
                s = jnp.int32(0)
                for j in range(_BLK):
                    s = extract(p, j, vec, tiles, rins, s)
                    m = j + _NBUF
                    if m < _BLK:
                        maybe_issue(m, tiles, cols)
                    elif nxt_io is not None:
                        maybe_issue(m - _BLK, tilesn, colsn)
                pltpu.async_copy(
                    rows_v.at[p], out_hbm.at[pl.ds(base + io, _BLK)],
                    semw[p])

            vec0 = idx_v[pl.ds(0, _BLK)]
            t0, c0, _ = lanes(vec0)
            for j in range(_NBUF):
                maybe_issue(j, t0, c0)

            block(0, 0, _BLK, skip_wait=first)
            block(_BLK, 1, 2 * _BLK, skip_wait=first)

            def steady(i0):
                block(i0, 0, i0 + _BLK)
                block(i0 + _BLK, 1, i0 + 2 * _BLK)

            pl.loop(2 * _BLK, b_per_w - 2 * _BLK, step=2 * _BLK)(steady)

            block(b_per_w - 2 * _BLK, 0, b_per_w - _BLK)
            block(b_per_w - _BLK, 1, None)

        run_table(wst_hbm, out_s_hbm, True)
        run_table(wat_hbm, out_a_hbm, False)

        for p in range(2):
            pltpu.make_async_copy(
                rows_v.at[p], out_a_hbm.at[pl.ds(base, _BLK)],
                semw[p]).wait()

    return gather


@functools.lru_cache(maxsize=None)
def _make_unpermute(B, D):
    info = plsc.get_sparse_core_info()
    nw = info.num_cores * info.num_subcores
    b_per_w = B // nw
    n_desc = b_per_w // _IDX
    mesh = plsc.VectorSubcoreMesh(core_axis_name="c", subcore_axis_name="s")

    @functools.partial(
        pl.kernel,
        mesh=mesh,
        out_type=(
            jax.ShapeDtypeStruct((B, D), jnp.float32),
            jax.ShapeDtypeStruct((B, D), jnp.float32),
        ),
        scratch_types=[
            pltpu.VMEM((n_desc, _IDX), jnp.int32),
            pltpu.VMEM((b_per_w, D), jnp.float32),
            pltpu.SemaphoreType.DMA,
        ],
        compiler_params=pltpu.CompilerParams(use_tc_tiling_on_sc=False),
    )
    def unpermute(rows_s_hbm, rows_a_hbm, perm_hbm, out_s_hbm, out_a_hbm,
                  idx_v, rows_v, sem):
        wid = lax.axis_index("s") * info.num_cores + lax.axis_index("c")
        base = wid * b_per_w
        pltpu.sync_copy(perm_hbm.at[wid], idx_v)

        def scatter(rows_hbm, out_hbm):
            pltpu.sync_copy(rows_hbm.at[pl.ds(base, b_per_w)], rows_v)
            copies = []
            for j in range(n_desc):
                src = rows_v.at[pl.ds(j * _IDX, _IDX)]
                copies.append(
                    pltpu.async_copy(src, out_hbm.at[idx_v.at[j]], sem))
            for c in copies:
                c.wait()

        scatter(rows_s_hbm, out_s_hbm)
        scatter(rows_a_hbm, out_a_hbm)

    return unpermute


def kernel(instance_ids, W_shape, W_appearance):
    (B,) = instance_ids.shape
    V, D = W_shape.shape
    info = plsc.get_sparse_core_info()
    nw = info.num_cores * info.num_subcores
    ids = instance_ids.astype(jnp.int32)
    sid, perm = lax.sort_key_val(ids, jnp.arange(B, dtype=jnp.int32))
    rows_s, rows_a = _make_gather(B, V, D)(sid, W_shape.T, W_appearance.T)
    perm3 = perm.reshape(nw, (B // nw) // _IDX, _IDX)
    out_s, out_a = _make_unpermute(B, D)(rows_s, rows_a, perm3)
    return (out_s, out_a)
